# baseline (speedup 1.0000x reference)
import jax
import jax.numpy as jnp
import numpy as np
from jax import lax
from jax.experimental import pallas as pl
from jax.experimental.pallas import tpu as pltpu
from jax.experimental.pallas import tpu_sc as plsc

F = 26
E = 16
B = 16384
ROWS_PER_FIELD = 38462
TOTAL_ROWS = F * ROWS_PER_FIELD
NW = 32
IW = 128
BW = B // NW
QB = BW // IW
NSTEP = F * QB
NRUN = F * 2 * (B // IW)

SLOTS = 8
LEAD = 4


def _sc_body(xt, emb_flat, fc_f, bias16, emb_p, lr_out,
             xt_v, gidf_v, idxb, pbuf, fcv, lr_v, bias_v,
             gsem, wsem, fcsem, ldsem):
    wid = lax.axis_index("s") * 2 + lax.axis_index("c")

    c1 = pltpu.async_copy(xt.at[:, pl.ds(wid * BW, BW)], xt_v, ldsem)
    c2 = pltpu.async_copy(bias16, bias_v, ldsem)
    c1.wait(); c2.wait()

    @pl.loop(0, F)
    def _(f):
        o = f * ROWS_PER_FIELD
        for q in range(QB):
            for c in range(IW // 16):
                gidf_v[f * QB + q, pl.ds(c * 16, 16)] = (
                    xt_v[f, pl.ds(q * IW + c * 16, 16)] + o)

    def g_desc(s, c):
        return pltpu.make_async_copy(emb_flat.at[idxb.at[s, c]],
                                     pbuf.at[s, c], gsem.at[s])

    def build_and_fire(j, s):
        for v in range(IW // 16):
            sl = pl.ds(v * 16, 16)
            gv = gidf_v[j, sl]
            for c in range(E):
                idxb[s, c, sl] = gv + c * TOTAL_ROWS
        for c in range(E):
            g_desc(s, c).start()

    def wait_gather(s):
        for c in range(E):
            g_desc(s, c).wait()

    def w_desc(j, half, s):
        f = lax.shift_right_logical(j, 2)
        q = lax.bitwise_and(j, 3)
        run = (f * 2 + half) * (B // IW) + wid * QB + q
        return pltpu.make_async_copy(pbuf.at[s, pl.ds(half * 8, 8)],
                                     emb_p.at[run], wsem.at[s])

    def fc_desc(j):
        return pltpu.make_async_copy(fc_f.at[gidf_v.at[j]],
                                     fcv.at[pl.ds(j * IW, IW)], fcsem)

    for s in range(LEAD):
        build_and_fire(s, s)

    @pl.loop(0, NSTEP // SLOTS)
    def _(g):
        for t in range(SLOTS):
            j = g * SLOTS + t
            wait_gather(t)
            fc_desc(j).start()

            @pl.when(j >= LEAD)
            def _():
                fc_desc(j - LEAD).wait()

            w_desc(j, 0, t).start()
            w_desc(j, 1, t).start()
            s2 = (t + LEAD) % SLOTS
            j2 = j + LEAD

            @pl.when(j2 >= SLOTS)
            def _():
                w_desc(j2 - SLOTS, 0, s2).wait()
                w_desc(j2 - SLOTS, 1, s2).wait()

            @pl.when(j2 < NSTEP)
            def _():
                build_and_fire(j2, s2)

    for j in range(NSTEP - LEAD, NSTEP):
        fc_desc(j).wait()

    @pl.loop(0, BW // 16)
    def _(g):
        acc = bias_v[...]
        for f in range(F):
            acc = acc + fcv[pl.ds(f * BW + g * 16, 16)]
        lr_v[pl.ds(g * 16, 16)] = acc

    pltpu.sync_copy(lr_v, lr_out.at[pl.ds(wid * BW, BW)])

    for j in range(NSTEP - LEAD, NSTEP):
        t = j % SLOTS
        w_desc(j, 0, t).wait()
        w_desc(j, 1, t).wait()


@jax.jit
def _sc_call(xt, emb_flat, fc_f, bias16):
    mesh = plsc.VectorSubcoreMesh(core_axis_name="c", subcore_axis_name="s")
    return pl.kernel(
        _sc_body,
        out_type=(
            jax.ShapeDtypeStruct((NRUN, 8, IW), jnp.float32),
            jax.ShapeDtypeStruct((B,), jnp.float32),
        ),
        mesh=mesh,
        compiler_params=pltpu.CompilerParams(use_tc_tiling_on_sc=False),
        scratch_types=[
            pltpu.VMEM((F, BW), jnp.int32),
            pltpu.VMEM((NSTEP, IW), jnp.int32),
            pltpu.VMEM((SLOTS, E, IW), jnp.int32),
            pltpu.VMEM((SLOTS, E, IW), jnp.float32),
            pltpu.VMEM((F * BW,), jnp.float32),
            pltpu.VMEM((BW,), jnp.float32),
            pltpu.VMEM((16,), jnp.float32),
            pltpu.SemaphoreType.DMA((SLOTS,)),
            pltpu.SemaphoreType.DMA((SLOTS,)),
            pltpu.SemaphoreType.DMA,
            pltpu.SemaphoreType.DMA,
        ],
    )(xt, emb_flat, fc_f, bias16)


def kernel(x, emb_table, fc_table, bias):
    xt = x.T
    emb_flat = emb_table.T.reshape(E * TOTAL_ROWS)
    fc_f = fc_table.reshape(TOTAL_ROWS)
    bias16 = jnp.broadcast_to(bias.astype(jnp.float32), (16,))
    emb_p, lr = _sc_call(xt, emb_flat, fc_f, bias16)
    emb = (emb_p.reshape(F, 2, B // IW, 8, IW)
           .transpose(2, 4, 0, 1, 3)
           .reshape(B, F, E))
    return emb, lr.reshape(B, 1)


# speedup vs baseline: 1.9155x; 1.9155x over previous
import jax
import jax.numpy as jnp
import numpy as np
from jax import lax
from jax.experimental import pallas as pl
from jax.experimental.pallas import tpu as pltpu
from jax.experimental.pallas import tpu_sc as plsc

F = 26
E = 16
B = 16384
ROWS_PER_FIELD = 38462
TOTAL_ROWS = F * ROWS_PER_FIELD
NW = 32
IW = 128
BW = B // NW
QB = BW // IW
NSTEP = F * QB
NRUN = F * 2 * (B // IW)

SLOTS = 8
LEAD = 4

CB = 1024
NB = (TOTAL_ROWS + CB - 1) // CB
FLAT_LEN = NB * CB * E


def _flat_kernel(x_ref, o_ref):
    for c in range(E):
        o_ref[pl.ds(c * CB, CB)] = x_ref[c, :]


@jax.jit
def _flatten_table(emb_t_t):
    return pl.pallas_call(
        _flat_kernel,
        grid=(NB,),
        in_specs=[pl.BlockSpec((E, CB), lambda i: (0, i))],
        out_specs=pl.BlockSpec((E * CB,), lambda i: (i,)),
        out_shape=jax.ShapeDtypeStruct((FLAT_LEN,), jnp.float32),
    )(emb_t_t)


def _sc_body(xt, emb_flat, fc_f, bias16, emb_p, lr_out,
             xt_v, gidf_v, idxb, pbuf, fcv, lr_v, bias_v,
             gsem, wsem, fcsem, ldsem):
    wid = lax.axis_index("s") * 2 + lax.axis_index("c")

    c1 = pltpu.async_copy(xt.at[:, pl.ds(wid * BW, BW)], xt_v, ldsem)
    c2 = pltpu.async_copy(bias16, bias_v, ldsem)
    c1.wait(); c2.wait()

    @pl.loop(0, F)
    def _(f):
        o = f * ROWS_PER_FIELD
        for q in range(QB):
            for c in range(IW // 16):
                gidf_v[f * QB + q, pl.ds(c * 16, 16)] = (
                    xt_v[f, pl.ds(q * IW + c * 16, 16)] + o)

    def g_desc(s, c):
        return pltpu.make_async_copy(emb_flat.at[idxb.at[s, c]],
                                     pbuf.at[s, c], gsem.at[s])

    def build_and_fire(j, s):
        for v in range(IW // 16):
            sl = pl.ds(v * 16, 16)
            gv = gidf_v[j, sl]
            base = lax.bitwise_or(
                lax.shift_left(lax.shift_right_logical(gv, 10), 14),
                lax.bitwise_and(gv, CB - 1))
            for c in range(E):
                idxb[s, c, sl] = base + c * CB
        for c in range(E):
            g_desc(s, c).start()

    def wait_gather(s):
        for c in range(E):
            g_desc(s, c).wait()

    def w_desc(j, half, s):
        f = lax.shift_right_logical(j, 2)
        q = lax.bitwise_and(j, 3)
        run = (f * 2 + half) * (B // IW) + wid * QB + q
        return pltpu.make_async_copy(pbuf.at[s, pl.ds(half * 8, 8)],
                                     emb_p.at[run], wsem.at[s])

    def fc_desc(j):
        return pltpu.make_async_copy(fc_f.at[gidf_v.at[j]],
                                     fcv.at[pl.ds(j * IW, IW)], fcsem)

    for s in range(LEAD):
        build_and_fire(s, s)

    @pl.loop(0, NSTEP // SLOTS)
    def _(g):
        for t in range(SLOTS):
            j = g * SLOTS + t
            wait_gather(t)
            fc_desc(j).start()

            @pl.when(j >= LEAD)
            def _():
                fc_desc(j - LEAD).wait()

            w_desc(j, 0, t).start()
            w_desc(j, 1, t).start()
            s2 = (t + LEAD) % SLOTS
            j2 = j + LEAD

            @pl.when(j2 >= SLOTS)
            def _():
                w_desc(j2 - SLOTS, 0, s2).wait()
                w_desc(j2 - SLOTS, 1, s2).wait()

            @pl.when(j2 < NSTEP)
            def _():
                build_and_fire(j2, s2)

    for j in range(NSTEP - LEAD, NSTEP):
        fc_desc(j).wait()

    @pl.loop(0, BW // 16)
    def _(g):
        acc = bias_v[...]
        for f in range(F):
            acc = acc + fcv[pl.ds(f * BW + g * 16, 16)]
        lr_v[pl.ds(g * 16, 16)] = acc

    pltpu.sync_copy(lr_v, lr_out.at[pl.ds(wid * BW, BW)])

    for j in range(NSTEP - LEAD, NSTEP):
        t = j % SLOTS
        w_desc(j, 0, t).wait()
        w_desc(j, 1, t).wait()


@jax.jit
def _sc_call(xt, emb_flat, fc_f, bias16):
    mesh = plsc.VectorSubcoreMesh(core_axis_name="c", subcore_axis_name="s")
    return pl.kernel(
        _sc_body,
        out_type=(
            jax.ShapeDtypeStruct((NRUN, 8, IW), jnp.float32),
            jax.ShapeDtypeStruct((B,), jnp.float32),
        ),
        name="ctr_gather",
        mesh=mesh,
        compiler_params=pltpu.CompilerParams(use_tc_tiling_on_sc=False),
        scratch_types=[
            pltpu.VMEM((F, BW), jnp.int32),
            pltpu.VMEM((NSTEP, IW), jnp.int32),
            pltpu.VMEM((SLOTS, E, IW), jnp.int32),
            pltpu.VMEM((SLOTS, E, IW), jnp.float32),
            pltpu.VMEM((F * BW,), jnp.float32),
            pltpu.VMEM((BW,), jnp.float32),
            pltpu.VMEM((16,), jnp.float32),
            pltpu.SemaphoreType.DMA((SLOTS,)),
            pltpu.SemaphoreType.DMA((SLOTS,)),
            pltpu.SemaphoreType.DMA,
            pltpu.SemaphoreType.DMA,
        ],
    )(xt, emb_flat, fc_f, bias16)


def kernel(x, emb_table, fc_table, bias):
    xt = x.T
    emb_flat = _flatten_table(emb_table.T)
    fc_f = fc_table.reshape(TOTAL_ROWS)
    bias16 = jnp.broadcast_to(bias.astype(jnp.float32), (16,))
    emb_p, lr = _sc_call(xt, emb_flat, fc_f, bias16)
    emb = (emb_p.reshape(F, 2, B // IW, 8, IW)
           .transpose(2, 4, 0, 1, 3)
           .reshape(B, F, E))
    return emb, lr.reshape(B, 1)


# speedup vs baseline: 3.7399x; 1.9524x over previous
import jax
import jax.numpy as jnp
import numpy as np
from jax import lax
from jax.experimental import pallas as pl
from jax.experimental.pallas import tpu as pltpu
from jax.experimental.pallas import tpu_sc as plsc

F = 26
E = 16
B = 16384
ROWS_PER_FIELD = 38462
TOTAL_ROWS = F * ROWS_PER_FIELD
NW = 32
IW = 128
BW = B // NW
QB = BW // IW
NSTEP = F * QB
NRUN = F * 2 * (B // IW)

SLOTS = 8
LEAD = 4

CB = 1024
FB = 8
NB = (TOTAL_ROWS + CB * FB - 1) // (CB * FB) * FB
FLAT_LEN = NB * CB * E


def _flat_kernel(x_ref, o_ref):
    for q in range(FB):
        for c in range(E):
            o_ref[pl.ds((q * E + c) * CB, CB)] = x_ref[c, pl.ds(q * CB, CB)]


@jax.jit
def _flatten_table(emb_t_t):
    return pl.pallas_call(
        _flat_kernel,
        grid=(NB // FB,),
        in_specs=[pl.BlockSpec((E, CB * FB), lambda i: (0, i))],
        out_specs=pl.BlockSpec((E * CB * FB,), lambda i: (i,)),
        out_shape=jax.ShapeDtypeStruct((FLAT_LEN,), jnp.float32),
    )(emb_t_t)


def _sc_body(xt, emb_flat, fc_f, bias16, emb_p, lr_out,
             xt_v, gidf_v, idxb, pbuf, fcv, lr_v, bias_v,
             gsem, wsem, fcsem, ldsem):
    wid = lax.axis_index("s") * 2 + lax.axis_index("c")

    c1 = pltpu.async_copy(xt.at[:, pl.ds(wid * BW, BW)], xt_v, ldsem)
    c2 = pltpu.async_copy(bias16, bias_v, ldsem)
    c1.wait(); c2.wait()

    @pl.loop(0, F)
    def _(f):
        o = f * ROWS_PER_FIELD
        for q in range(QB):
            for c in range(IW // 16):
                gidf_v[f * QB + q, pl.ds(c * 16, 16)] = (
                    xt_v[f, pl.ds(q * IW + c * 16, 16)] + o)

    def g_desc(s, c):
        return pltpu.make_async_copy(emb_flat.at[idxb.at[s, c]],
                                     pbuf.at[s, c], gsem.at[s])

    def build_and_fire(j, s):
        for v in range(IW // 16):
            sl = pl.ds(v * 16, 16)
            gv = gidf_v[j, sl]
            base = lax.bitwise_or(
                lax.shift_left(lax.shift_right_logical(gv, 10), 14),
                lax.bitwise_and(gv, CB - 1))
            for c in range(E):
                idxb[s, c, sl] = base + c * CB
        for c in range(E):
            g_desc(s, c).start()

    def wait_gather(s):
        for c in range(E):
            g_desc(s, c).wait()

    def w_desc(j, half, s):
        f = lax.shift_right_logical(j, 2)
        q = lax.bitwise_and(j, 3)
        run = (f * 2 + half) * (B // IW) + wid * QB + q
        return pltpu.make_async_copy(pbuf.at[s, pl.ds(half * 8, 8)],
                                     emb_p.at[run], wsem.at[s])

    def fc_desc(j):
        return pltpu.make_async_copy(fc_f.at[gidf_v.at[j]],
                                     fcv.at[pl.ds(j * IW, IW)], fcsem)

    for s in range(LEAD):
        build_and_fire(s, s)

    @pl.loop(0, NSTEP // SLOTS)
    def _(g):
        for t in range(SLOTS):
            j = g * SLOTS + t
            wait_gather(t)
            fc_desc(j).start()

            @pl.when(j >= LEAD)
            def _():
                fc_desc(j - LEAD).wait()

            w_desc(j, 0, t).start()
            w_desc(j, 1, t).start()
            s2 = (t + LEAD) % SLOTS
            j2 = j + LEAD

            @pl.when(j2 >= SLOTS)
            def _():
                w_desc(j2 - SLOTS, 0, s2).wait()
                w_desc(j2 - SLOTS, 1, s2).wait()

            @pl.when(j2 < NSTEP)
            def _():
                build_and_fire(j2, s2)

    for j in range(NSTEP - LEAD, NSTEP):
        fc_desc(j).wait()

    @pl.loop(0, BW // 16)
    def _(g):
        acc = bias_v[...]
        for f in range(F):
            acc = acc + fcv[pl.ds(f * BW + g * 16, 16)]
        lr_v[pl.ds(g * 16, 16)] = acc

    pltpu.sync_copy(lr_v, lr_out.at[pl.ds(wid * BW, BW)])

    for j in range(NSTEP - LEAD, NSTEP):
        t = j % SLOTS
        w_desc(j, 0, t).wait()
        w_desc(j, 1, t).wait()


@jax.jit
def _sc_call(xt, emb_flat, fc_f, bias16):
    mesh = plsc.VectorSubcoreMesh(core_axis_name="c", subcore_axis_name="s")
    return pl.kernel(
        _sc_body,
        out_type=(
            jax.ShapeDtypeStruct((NRUN, 8, IW), jnp.float32),
            jax.ShapeDtypeStruct((B,), jnp.float32),
        ),
        name="ctr_gather",
        mesh=mesh,
        compiler_params=pltpu.CompilerParams(use_tc_tiling_on_sc=False),
        scratch_types=[
            pltpu.VMEM((F, BW), jnp.int32),
            pltpu.VMEM((NSTEP, IW), jnp.int32),
            pltpu.VMEM((SLOTS, E, IW), jnp.int32),
            pltpu.VMEM((SLOTS, E, IW), jnp.float32),
            pltpu.VMEM((F * BW,), jnp.float32),
            pltpu.VMEM((BW,), jnp.float32),
            pltpu.VMEM((16,), jnp.float32),
            pltpu.SemaphoreType.DMA((SLOTS,)),
            pltpu.SemaphoreType.DMA((SLOTS,)),
            pltpu.SemaphoreType.DMA,
            pltpu.SemaphoreType.DMA,
        ],
    )(xt, emb_flat, fc_f, bias16)


def kernel(x, emb_table, fc_table, bias):
    xt = x.T
    emb_flat = _flatten_table(emb_table.T)
    fc_f = fc_table.reshape(TOTAL_ROWS)
    bias16 = jnp.broadcast_to(bias.astype(jnp.float32), (16,))
    emb_p, lr = _sc_call(xt, emb_flat, fc_f, bias16)
    emb = (emb_p.reshape(F, 2, B // IW, 8, IW)
           .transpose(2, 4, 0, 1, 3)
           .reshape(B, F, E))
    return emb, lr.reshape(B, 1)


# speedup vs baseline: 3.9871x; 1.0661x over previous
import jax
import jax.numpy as jnp
import numpy as np
from jax import lax
from jax.experimental import pallas as pl
from jax.experimental.pallas import tpu as pltpu
from jax.experimental.pallas import tpu_sc as plsc

F = 26
E = 16
B = 16384
ROWS_PER_FIELD = 38462
TOTAL_ROWS = F * ROWS_PER_FIELD
NW = 32
IW = 128
BW = B // NW
QB = BW // IW
NSTEP = F * QB
NRUN = F * 2 * (B // IW)

SLOTS = 8
LEAD = 4

CB = 1024
FB = 16
NB = (TOTAL_ROWS + CB * FB - 1) // (CB * FB) * FB
FLAT_LEN = NB * CB * E


def _flat_kernel(x_ref, o_ref):
    for q in range(FB):
        for c in range(E):
            o_ref[pl.ds((q * E + c) * CB, CB)] = x_ref[c, pl.ds(q * CB, CB)]


@jax.jit
def _flatten_table(emb_t_t):
    return pl.pallas_call(
        _flat_kernel,
        grid=(NB // FB,),
        in_specs=[pl.BlockSpec((E, CB * FB), lambda i: (0, i))],
        out_specs=pl.BlockSpec((E * CB * FB,), lambda i: (i,)),
        out_shape=jax.ShapeDtypeStruct((FLAT_LEN,), jnp.float32),
    )(emb_t_t)


def _sc_body(xt, emb_flat, fc_f, bias16, emb_p, lr_out,
             xt_v, gidf_v, idxb, pbuf, fcv, lr_v, bias_v,
             gsem, wsem, fcsem, ldsem):
    wid = lax.axis_index("s") * 2 + lax.axis_index("c")

    c1 = pltpu.async_copy(xt.at[:, pl.ds(wid * BW, BW)], xt_v, ldsem)
    c2 = pltpu.async_copy(bias16, bias_v, ldsem)
    c1.wait(); c2.wait()

    @pl.loop(0, F)
    def _(f):
        o = f * ROWS_PER_FIELD
        for q in range(QB):
            for c in range(IW // 16):
                gidf_v[f * QB + q, pl.ds(c * 16, 16)] = (
                    xt_v[f, pl.ds(q * IW + c * 16, 16)] + o)

    def g_desc(s, c):
        return pltpu.make_async_copy(emb_flat.at[idxb.at[s, c]],
                                     pbuf.at[s, c], gsem.at[s])

    def build_and_fire(j, s):
        for v in range(IW // 16):
            sl = pl.ds(v * 16, 16)
            gv = gidf_v[j, sl]
            base = lax.bitwise_or(
                lax.shift_left(lax.shift_right_logical(gv, 10), 14),
                lax.bitwise_and(gv, CB - 1))
            for c in range(E):
                idxb[s, c, sl] = base + c * CB
        for c in range(E):
            g_desc(s, c).start()

    def wait_gather(s):
        for c in range(E):
            g_desc(s, c).wait()

    def w_desc(j, half, s):
        f = lax.shift_right_logical(j, 2)
        q = lax.bitwise_and(j, 3)
        run = (f * 2 + half) * (B // IW) + wid * QB + q
        return pltpu.make_async_copy(pbuf.at[s, pl.ds(half * 8, 8)],
                                     emb_p.at[run], wsem.at[s])

    def fc_desc(j):
        return pltpu.make_async_copy(fc_f.at[gidf_v.at[j]],
                                     fcv.at[pl.ds(j * IW, IW)], fcsem)

    for s in range(LEAD):
        build_and_fire(s, s)

    @pl.loop(0, NSTEP // SLOTS)
    def _(g):
        for t in range(SLOTS):
            j = g * SLOTS + t
            wait_gather(t)
            fc_desc(j).start()

            @pl.when(j >= LEAD)
            def _():
                fc_desc(j - LEAD).wait()

            w_desc(j, 0, t).start()
            w_desc(j, 1, t).start()
            s2 = (t + LEAD) % SLOTS
            j2 = j + LEAD

            @pl.when(j2 >= SLOTS)
            def _():
                w_desc(j2 - SLOTS, 0, s2).wait()
                w_desc(j2 - SLOTS, 1, s2).wait()

            @pl.when(j2 < NSTEP)
            def _():
                build_and_fire(j2, s2)

    for j in range(NSTEP - LEAD, NSTEP):
        fc_desc(j).wait()

    @pl.loop(0, BW // 16)
    def _(g):
        acc = bias_v[...]
        for f in range(F):
            acc = acc + fcv[pl.ds(f * BW + g * 16, 16)]
        lr_v[pl.ds(g * 16, 16)] = acc

    pltpu.sync_copy(lr_v, lr_out.at[pl.ds(wid * BW, BW)])

    for j in range(NSTEP - LEAD, NSTEP):
        t = j % SLOTS
        w_desc(j, 0, t).wait()
        w_desc(j, 1, t).wait()


@jax.jit
def _sc_call(xt, emb_flat, fc_f, bias16):
    mesh = plsc.VectorSubcoreMesh(core_axis_name="c", subcore_axis_name="s")
    return pl.kernel(
        _sc_body,
        out_type=(
            jax.ShapeDtypeStruct((NRUN, 8, IW), jnp.float32),
            jax.ShapeDtypeStruct((B,), jnp.float32),
        ),
        name="ctr_gather",
        mesh=mesh,
        compiler_params=pltpu.CompilerParams(use_tc_tiling_on_sc=False),
        scratch_types=[
            pltpu.VMEM((F, BW), jnp.int32),
            pltpu.VMEM((NSTEP, IW), jnp.int32),
            pltpu.VMEM((SLOTS, E, IW), jnp.int32),
            pltpu.VMEM((SLOTS, E, IW), jnp.float32),
            pltpu.VMEM((F * BW,), jnp.float32),
            pltpu.VMEM((BW,), jnp.float32),
            pltpu.VMEM((16,), jnp.float32),
            pltpu.SemaphoreType.DMA((SLOTS,)),
            pltpu.SemaphoreType.DMA((SLOTS,)),
            pltpu.SemaphoreType.DMA,
            pltpu.SemaphoreType.DMA,
        ],
    )(xt, emb_flat, fc_f, bias16)


def kernel(x, emb_table, fc_table, bias):
    xt = x.T
    emb_flat = _flatten_table(emb_table.T)
    fc_f = fc_table[:, 0]
    bias16 = jnp.broadcast_to(bias.astype(jnp.float32), (16,))
    emb_p, lr = _sc_call(xt, emb_flat, fc_f, bias16)
    emb = (emb_p.reshape(F, 2, B // IW, 8, IW)
           .transpose(2, 4, 0, 1, 3)
           .reshape(B, F, E))
    return emb, lr.reshape(B, 1)
